# baseline (speedup 1.0000x reference)
import functools

import jax
import jax.numpy as jnp
from jax import lax
from jax.experimental import pallas as pl
from jax.experimental.pallas import tpu as pltpu
from jax.experimental.pallas import tpu_sc as plsc

L = 32
NF = 50000
NE = 10000
EM = 800000
EW = 200000
OUT = 3

NC = 2
NS = 16
NW = NC * NS
BLK = 128

NB_M = 196
NB_W = 49
EMP = NW * NB_M * BLK
EWP = NW * NB_W * BLK
NB_ALL = 2 * NB_M + NB_W
GM = NW * NB_ALL * BLK

NFP = 50048
NEP = 10048
STRIPE = NFP // NS

ROW_T = 4096


def _pad_rows(x, n):
    return jnp.pad(x, ((0, n - x.shape[0]), (0, 0)))


def _pad_idx(x, n, base, span):
    m = x.shape[0]
    pad = base + jnp.arange(n - m, dtype=jnp.int32) % span
    return jnp.concatenate([x, pad])


def _dot(a, b):
    return jnp.dot(a, b, preferred_element_type=jnp.float32)


def _enc_body(x_ref, w1_ref, w2_ref, w3_ref, mb_ref, v_ref, o_ref):
    x = _dot(x_ref[...], w1_ref[...]) + v_ref[0:1, :]
    x = jnp.maximum(x, 0.0)
    x = jnp.maximum(_dot(x, w2_ref[...]) + v_ref[1:2, :], 0.0)
    x = _dot(x, w3_ref[...]) + v_ref[2:3, :]
    o_ref[...] = _ln_p(x, mb_ref[...], v_ref[3:4, :], v_ref[4:5, :])


def _encoder(p, x, blk, mb):
    n, f4 = x.shape
    mlp = p["mlp"]
    full = lambda s: pl.BlockSpec(s, lambda i: (0, 0))
    return pl.pallas_call(
        _enc_body,
        grid=(n // blk,),
        in_specs=[
            pl.BlockSpec((blk, f4), lambda i: (i, 0)),
            full((f4, LP)), full((LP, LP)), full((LP, LP)),
            full((LP, LP)), full((8, LP)),
        ],
        out_specs=pl.BlockSpec((blk, LP), lambda i: (i, 0)),
        out_shape=jax.ShapeDtypeStruct((n, LP), jnp.float32),
        compiler_params=pltpu.CompilerParams(
            dimension_semantics=("arbitrary",)),
    )(x, _blk4(mlp[0]["W"]), _blk4(mlp[1]["W"]), _blk4(mlp[2]["W"]),
      mb, _pack_vecs_p(p))


LP = 4 * L
RT_P = ROW_T // 4


def _blk4(w):
    return jnp.kron(jnp.eye(4, dtype=jnp.float32), w)


def _mean_mat():
    return _blk4(jnp.full((L, L), 1.0 / L, jnp.float32))


def _pack_vecs_p(p):
    mlp = p["mlp"]
    rows = [jnp.tile(mlp[i]["b"], 4) for i in range(3)]
    rows.append(jnp.tile(p["ln_g"], 4))
    rows.append(jnp.tile(p["ln_b"], 4))
    rows += [jnp.zeros((LP,), jnp.float32)] * 3
    return jnp.stack(rows)


def _ln_p(x, mb, g, b):
    m = _dot(x, mb)
    d = x - m
    v = _dot(d * d, mb)
    return d * lax.rsqrt(v + 1e-5) * g + b


def _edge_body_p(e_ref, s_ref, d_ref, w1a_ref, w1b_ref, w1c_ref,
                 w2_ref, w3_ref, mb_ref, v_ref, new_ref, res_ref):
    e = e_ref[...]
    x = (_dot(e, w1a_ref[...])
         + _dot(s_ref[...], w1b_ref[...])
         + _dot(d_ref[...], w1c_ref[...])
         + v_ref[0:1, :])
    x = jnp.maximum(x, 0.0)
    x = jnp.maximum(_dot(x, w2_ref[...]) + v_ref[1:2, :], 0.0)
    x = _dot(x, w3_ref[...]) + v_ref[2:3, :]
    x = _ln_p(x, mb_ref[...], v_ref[3:4, :], v_ref[4:5, :])
    new_ref[...] = x
    res_ref[...] = e + x


def _edge_mlp(p, e, src, dst, mb):
    n = e.shape[0]
    grid = n // RT_P
    mlp = p["mlp"]
    full = lambda s: pl.BlockSpec(s, lambda i: (0, 0))
    (sa, so), (da, do) = src, dst
    w1 = mlp[0]["W"]
    return pl.pallas_call(
        _edge_body_p,
        grid=(grid,),
        in_specs=[
            pl.BlockSpec((RT_P, LP), lambda i: (i, 0)),
            pl.BlockSpec((RT_P, LP), lambda i, _o=so: (i + _o, 0)),
            pl.BlockSpec((RT_P, LP), lambda i, _o=do: (i + _o, 0)),
            full((LP, LP)), full((LP, LP)), full((LP, LP)),
            full((LP, LP)), full((LP, LP)), full((LP, LP)), full((8, LP)),
        ],
        out_specs=[pl.BlockSpec((RT_P, LP), lambda i: (i, 0))] * 2,
        out_shape=[jax.ShapeDtypeStruct((n, LP), jnp.float32)] * 2,
        compiler_params=pltpu.CompilerParams(
            dimension_semantics=("arbitrary",)),
    )(e, sa, da, _blk4(w1[0:L]),
      _blk4(w1[L:2 * L]).astype(jnp.bfloat16),
      _blk4(w1[2 * L:]).astype(jnp.bfloat16),
      _blk4(mlp[1]["W"]), _blk4(mlp[2]["W"]), mb, _pack_vecs_p(p))


def _node_body_p(f_ref, am_ref, aw_ref, w1a_ref, w1b_ref, w1c_ref,
                 w2_ref, w3_ref, mb_ref, v_ref, o_ref):
    f = f_ref[...]
    x = (_dot(f, w1a_ref[...])
         + _dot(am_ref[0] + am_ref[1], w1b_ref[...])
         + _dot(aw_ref[0] + aw_ref[1], w1c_ref[...])
         + v_ref[0:1, :])
    x = jnp.maximum(x, 0.0)
    x = jnp.maximum(_dot(x, w2_ref[...]) + v_ref[1:2, :], 0.0)
    x = _dot(x, w3_ref[...]) + v_ref[2:3, :]
    o_ref[...] = f + _ln_p(x, mb_ref[...], v_ref[3:4, :], v_ref[4:5, :])


def _node_mlp(p, fl, amp, awp, mb):
    blk = 3128
    npk = NFP // 4
    grid = npk // blk
    mlp = p["mlp"]
    full = lambda s: pl.BlockSpec(s, lambda i: (0, 0))
    w1 = mlp[0]["W"]
    return pl.pallas_call(
        _node_body_p,
        grid=(grid,),
        in_specs=[
            pl.BlockSpec((blk, LP), lambda i: (i, 0)),
            pl.BlockSpec((2, blk, LP), lambda i: (0, i, 0)),
            pl.BlockSpec((2, blk, LP), lambda i: (0, i, 0)),
            full((LP, LP)), full((LP, LP)), full((LP, LP)),
            full((LP, LP)), full((LP, LP)), full((LP, LP)), full((8, LP)),
        ],
        out_specs=pl.BlockSpec((blk, LP), lambda i: (i, 0)),
        out_shape=jax.ShapeDtypeStruct((npk, LP), jnp.float32),
        compiler_params=pltpu.CompilerParams(
            dimension_semantics=("arbitrary",)),
    )(fl, amp, awp, _blk4(w1[0:L]), _blk4(w1[L:2 * L]), _blk4(w1[2 * L:]),
      _blk4(mlp[1]["W"]), _blk4(mlp[2]["W"]), mb, _pack_vecs_p(p))


def _dec_body(x_ref, w1_ref, w2_ref, w3_ref, v_ref, o_ref):
    x = jnp.maximum(_dot(x_ref[...], w1_ref[...]) + v_ref[0:1, :], 0.0)
    x = jnp.maximum(_dot(x, w2_ref[...]) + v_ref[1:2, :], 0.0)
    o_ref[...] = _dot(x, w3_ref[...]) + v_ref[2:3, 0:4 * OUT]


def _decoder(p, fl):
    blk = 3128
    npk = NFP // 4
    mlp = p["mlp"]
    full = lambda s: pl.BlockSpec(s, lambda i: (0, 0))
    b3 = jnp.tile(mlp[2]["b"], 4)
    vecs = jnp.stack([jnp.tile(mlp[0]["b"], 4), jnp.tile(mlp[1]["b"], 4),
                      jnp.pad(b3, (0, LP - b3.shape[0]))]
                     + [jnp.zeros((LP,), jnp.float32)] * 5)
    return pl.pallas_call(
        _dec_body,
        grid=(npk // blk,),
        in_specs=[
            pl.BlockSpec((blk, LP), lambda i: (i, 0)),
            full((LP, LP)), full((LP, LP)), full((LP, 4 * OUT)),
            full((8, LP)),
        ],
        out_specs=pl.BlockSpec((blk, 4 * OUT), lambda i: (i, 0)),
        out_shape=jax.ShapeDtypeStruct((npk, 4 * OUT), jnp.float32),
        compiler_params=pltpu.CompilerParams(
            dimension_semantics=("arbitrary",)),
    )(fl, _blk4(mlp[0]["W"]), _blk4(mlp[1]["W"]), _blk4(mlp[2]["W"]), vecs)


@functools.cache
def _sc_mesh():
    return plsc.VectorSubcoreMesh(
        core_axis_name="c", subcore_axis_name="s",
        num_cores=NC, num_subcores=NS)


NK = 7


def _gather_call(table, idx3, nb):
    rows_pw = nb * BLK
    n = NW * rows_pw
    ng = nb // NK
    trows = table.shape[0]
    tstripe = trows // NS
    dt = table.dtype

    @functools.partial(
        pl.kernel,
        out_type=jax.ShapeDtypeStruct((n, L), dt),
        mesh=_sc_mesh(),
        compiler_params=pltpu.CompilerParams(use_tc_tiling_on_sc=False),
        scratch_types=[
            pltpu.VMEM((NK, BLK), jnp.int32),
            pltpu.VMEM((NK, BLK, L), dt),
            pltpu.VMEM_SHARED((trows, L), dt),
            pltpu.SemaphoreType.DMA,
            pltpu.SemaphoreType.DMA,
        ],
    )
    def k(table_ref, idx_ref, out_ref, idx_v, bufs, tab, semg, sems):
        c = lax.axis_index("c")
        s = lax.axis_index("s")
        wid = s * NC + c
        base = wid * rows_pw
        pltpu.sync_copy(table_ref.at[pl.ds(s * tstripe, tstripe)],
                        tab.at[pl.ds(s * tstripe, tstripe)])
        plsc.subcore_barrier()

        def body(g, _):
            b0 = g * NK
            di = pltpu.async_copy(idx_ref.at[wid, pl.ds(b0, NK)], idx_v,
                                  semg)
            di.wait()
            ds = [pltpu.async_copy(tab.at[idx_v.at[j]],
                                   bufs.at[j], semg) for j in range(NK)]
            for d in ds:
                d.wait()
            ss = [pltpu.async_copy(
                bufs.at[j],
                out_ref.at[pl.ds(base + (b0 + j) * BLK, BLK)], sems)
                for j in range(NK)]
            for d in ss:
                d.wait()
            return 0

        lax.fori_loop(0, ng, body, 0)
        plsc.subcore_barrier()

    return k(table, idx3)


def _scatter_call(vals, idx3, zeros, nb):
    rows_pw = nb * BLK

    @functools.partial(
        pl.kernel,
        out_type=jax.ShapeDtypeStruct((NC, NFP, L), jnp.float32),
        mesh=_sc_mesh(),
        compiler_params=pltpu.CompilerParams(use_tc_tiling_on_sc=False),
        scratch_types=[
            pltpu.VMEM((NK, BLK), jnp.int32),
            pltpu.VMEM((NK, BLK, L), jnp.float32),
            pltpu.VMEM_SHARED((NFP, L), jnp.float32),
            pltpu.SemaphoreType.DMA,
            pltpu.SemaphoreType.DMA,
        ],
    )
    def k(vals_ref, idx_ref, z_ref, out_ref, idx_v, bufs, acc, semr, semw):
        c = lax.axis_index("c")
        s = lax.axis_index("s")
        wid = s * NC + c
        base = wid * rows_pw
        ng = nb // NK
        pltpu.sync_copy(z_ref.at[pl.ds(s * STRIPE, STRIPE)],
                        acc.at[pl.ds(s * STRIPE, STRIPE)])
        plsc.subcore_barrier()

        def body(g, _):
            b0 = g * NK
            di = pltpu.async_copy(idx_ref.at[wid, pl.ds(b0, NK)], idx_v,
                                  semr)
            ds = [pltpu.async_copy(
                vals_ref.at[pl.ds(base + (b0 + j) * BLK, BLK)],
                bufs.at[j], semr) for j in range(NK)]
            di.wait()
            for d in ds:
                d.wait()
            ss = [pltpu.async_copy(bufs.at[j], acc.at[idx_v.at[j]],
                                   semw, add=True) for j in range(NK)]
            for d in ss:
                d.wait()
            return 0

        lax.fori_loop(0, ng, body, 0)
        plsc.subcore_barrier()
        pltpu.sync_copy(acc.at[pl.ds(s * STRIPE, STRIPE)],
                        out_ref.at[c, pl.ds(s * STRIPE, STRIPE)])

    return k(vals, idx3, zeros)


def kernel(fluid_node_attr, env_node_attr, mesh_edge_attr, world_edge_attr,
           params, mesh_edge_index, world_edge_index):
    p = params
    fl_attr = _pad_rows(fluid_node_attr, NFP)
    env_attr = _pad_rows(env_node_attr, NEP)
    me_attr = _pad_rows(mesh_edge_attr, EMP)
    we_attr = _pad_rows(world_edge_attr, EWP)

    ms = _pad_idx(mesh_edge_index[0], EMP, 0, NF)
    md = _pad_idx(mesh_edge_index[1], EMP, NF, NFP - NF)
    ws = _pad_idx(world_edge_index[0], EWP, 0, NE)
    wd = _pad_idx(world_edge_index[1], EWP, NF, NFP - NF)
    gidx = jnp.concatenate([ms, md, wd]).reshape(NW, NB_ALL, BLK)
    ws3 = ws.reshape(NW, NB_W, BLK)
    md3 = md.reshape(NW, NB_M, BLK)
    wd3 = wd.reshape(NW, NB_W, BLK)
    zeros_nf = jnp.zeros((NFP, L), jnp.float32)

    pk = lambda x: x.reshape(-1, LP)
    unpk = lambda x: x.reshape(-1, L)
    pk3 = lambda x: x.reshape(2, -1, LP)
    b16 = lambda x: unpk(x).astype(jnp.bfloat16)
    mb = _mean_mat()

    fl_p = _encoder(p["node_enc"], fl_attr.reshape(-1, 48), 3128, mb)
    el_p = _encoder(p["node_enc"], env_attr.reshape(-1, 48), 2512, mb)
    me_p = _encoder(p["mesh_enc"], me_attr.reshape(-1, 28), RT_P, mb)
    we_p = _encoder(p["world_enc"], we_attr.reshape(-1, 16), RT_P, mb)

    gws = _gather_call(b16(el_p), ws3, NB_W)
    gws_p = pk(gws)
    nbm = EMP // ROW_T
    for sp in p["steps"]:
        g = _gather_call(b16(fl_p), gidx, NB_ALL)
        gp = pk(g)
        mnew_p, me_p = _edge_mlp(sp["mesh_edge"], me_p, (gp, 0),
                                 (gp, nbm), mb)
        wnew_p, we_p = _edge_mlp(sp["world_edge"], we_p, (gws_p, 0),
                                 (gp, 2 * nbm), mb)
        amp = _scatter_call(unpk(mnew_p), md3, zeros_nf, NB_M)
        awp = _scatter_call(unpk(wnew_p), wd3, zeros_nf, NB_W)
        fl_p = _node_mlp(sp["node"], fl_p, pk3(amp), pk3(awp), mb)

    return _decoder(p["decoder"], fl_p).reshape(-1, OUT)[:NF]


# speedup vs baseline: 1.4081x; 1.4081x over previous
import functools

import jax
import jax.numpy as jnp
from jax import lax
from jax.experimental import pallas as pl
from jax.experimental.pallas import tpu as pltpu
from jax.experimental.pallas import tpu_sc as plsc

L = 32
NF = 50000
NE = 10000
EM = 800000
EW = 200000
OUT = 3

NC = 2
NS = 16
NW = NC * NS
BLK = 128

NB_M = 196
NB_W = 49
EMP = NW * NB_M * BLK
EWP = NW * NB_W * BLK
NB_ALL = 2 * NB_M + NB_W
GM = NW * NB_ALL * BLK

NFP = 50048
NEP = 10048
STRIPE = NFP // NS

ROW_T = 4096


def _pad_rows(x, n):
    return jnp.pad(x, ((0, n - x.shape[0]), (0, 0)))


def _pad_idx(x, n, base, span):
    m = x.shape[0]
    pad = base + jnp.arange(n - m, dtype=jnp.int32) % span
    return jnp.concatenate([x, pad])


def _dot(a, b):
    return jnp.dot(a, b, preferred_element_type=jnp.float32)


def _enc_body(x_ref, w1_ref, w2_ref, w3_ref, mb_ref, v_ref, o_ref):
    x = _dot(x_ref[...], w1_ref[...]) + v_ref[0:1, :]
    x = jnp.maximum(x, 0.0)
    x = jnp.maximum(_dot(x, w2_ref[...]) + v_ref[1:2, :], 0.0)
    x = _dot(x, w3_ref[...]) + v_ref[2:3, :]
    o_ref[...] = _ln_p(x, mb_ref[...], v_ref[3:4, :], v_ref[4:5, :])


def _encoder(p, x, blk, mb):
    n, f4 = x.shape
    mlp = p["mlp"]
    full = lambda s: pl.BlockSpec(s, lambda i: (0, 0))
    return pl.pallas_call(
        _enc_body,
        grid=(n // blk,),
        in_specs=[
            pl.BlockSpec((blk, f4), lambda i: (i, 0)),
            full((f4, LP)), full((LP, LP)), full((LP, LP)),
            full((LP, LP)), full((8, LP)),
        ],
        out_specs=pl.BlockSpec((blk, LP), lambda i: (i, 0)),
        out_shape=jax.ShapeDtypeStruct((n, LP), jnp.float32),
        compiler_params=pltpu.CompilerParams(
            dimension_semantics=("arbitrary",)),
    )(x, _blk4(mlp[0]["W"]), _blk4(mlp[1]["W"]), _blk4(mlp[2]["W"]),
      mb, _pack_vecs_p(p))


LP = 4 * L
RT_P = ROW_T // 4


def _blk4(w):
    return jnp.kron(jnp.eye(4, dtype=jnp.float32), w)


def _mean_mat():
    return _blk4(jnp.full((L, L), 1.0 / L, jnp.float32))


def _pack_vecs_p(p):
    mlp = p["mlp"]
    rows = [jnp.tile(mlp[i]["b"], 4) for i in range(3)]
    rows.append(jnp.tile(p["ln_g"], 4))
    rows.append(jnp.tile(p["ln_b"], 4))
    rows += [jnp.zeros((LP,), jnp.float32)] * 3
    return jnp.stack(rows)


def _ln_p(x, mb, g, b):
    m = _dot(x, mb)
    d = x - m
    v = _dot(d * d, mb)
    return d * lax.rsqrt(v + 1e-5) * g + b


def _edge_body_p(e_ref, s_ref, d_ref, w1a_ref, w1b_ref, w1c_ref,
                 w2_ref, w3_ref, mb_ref, v_ref, wn_ref, new_ref, res_ref):
    e = e_ref[...]
    x = (_dot(e, w1a_ref[...])
         + _dot(s_ref[...], w1b_ref[...])
         + _dot(d_ref[...], w1c_ref[...])
         + v_ref[0:1, :])
    x = jnp.maximum(x, 0.0)
    x = jnp.maximum(_dot(x, w2_ref[...]) + v_ref[1:2, :], 0.0)
    x = _dot(x, w3_ref[...]) + v_ref[2:3, :]
    x = _ln_p(x, mb_ref[...], v_ref[3:4, :], v_ref[4:5, :])
    new_ref[...] = _dot(x, wn_ref[...])
    res_ref[...] = e + x


def _edge_mlp(p, e, src, dst, mb, wn):
    n = e.shape[0]
    grid = n // RT_P
    mlp = p["mlp"]
    full = lambda s: pl.BlockSpec(s, lambda i: (0, 0))
    (sa, so), (da, do) = src, dst
    w1 = mlp[0]["W"]
    return pl.pallas_call(
        _edge_body_p,
        grid=(grid,),
        in_specs=[
            pl.BlockSpec((RT_P, LP), lambda i: (i, 0)),
            pl.BlockSpec((RT_P, LP), lambda i, _o=so: (i + _o, 0)),
            pl.BlockSpec((RT_P, LP), lambda i, _o=do: (i + _o, 0)),
            full((LP, LP)), full((LP, LP)), full((LP, LP)),
            full((LP, LP)), full((LP, LP)), full((LP, LP)), full((8, LP)),
            full((LP, LP)),
        ],
        out_specs=[pl.BlockSpec((RT_P, LP), lambda i: (i, 0))] * 2,
        out_shape=[jax.ShapeDtypeStruct((n, LP), jnp.float32)] * 2,
        compiler_params=pltpu.CompilerParams(
            dimension_semantics=("arbitrary",)),
    )(e, sa, da, _blk4(w1[0:L]), _blk4(w1[L:2 * L]), _blk4(w1[2 * L:]),
      _blk4(mlp[1]["W"]), _blk4(mlp[2]["W"]), mb, _pack_vecs_p(p), wn)


def _node_body_p(f_ref, a_ref, w1a_ref, w2_ref, w3_ref, mb_ref, v_ref,
                 o_ref):
    f = f_ref[...]
    x = _dot(f, w1a_ref[...]) + (a_ref[0] + a_ref[1]) + v_ref[0:1, :]
    x = jnp.maximum(x, 0.0)
    x = jnp.maximum(_dot(x, w2_ref[...]) + v_ref[1:2, :], 0.0)
    x = _dot(x, w3_ref[...]) + v_ref[2:3, :]
    o_ref[...] = f + _ln_p(x, mb_ref[...], v_ref[3:4, :], v_ref[4:5, :])


def _node_mlp(p, fl, ap, mb):
    blk = 3128
    npk = NFP // 4
    grid = npk // blk
    mlp = p["mlp"]
    full = lambda s: pl.BlockSpec(s, lambda i: (0, 0))
    w1 = mlp[0]["W"]
    return pl.pallas_call(
        _node_body_p,
        grid=(grid,),
        in_specs=[
            pl.BlockSpec((blk, LP), lambda i: (i, 0)),
            pl.BlockSpec((2, blk, LP), lambda i: (0, i, 0)),
            full((LP, LP)), full((LP, LP)), full((LP, LP)),
            full((LP, LP)), full((8, LP)),
        ],
        out_specs=pl.BlockSpec((blk, LP), lambda i: (i, 0)),
        out_shape=jax.ShapeDtypeStruct((npk, LP), jnp.float32),
        compiler_params=pltpu.CompilerParams(
            dimension_semantics=("arbitrary",)),
    )(fl, ap, _blk4(w1[0:L]),
      _blk4(mlp[1]["W"]), _blk4(mlp[2]["W"]), mb, _pack_vecs_p(p))


def _dec_body(x_ref, w1_ref, w2_ref, w3_ref, v_ref, o_ref):
    x = jnp.maximum(_dot(x_ref[...], w1_ref[...]) + v_ref[0:1, :], 0.0)
    x = jnp.maximum(_dot(x, w2_ref[...]) + v_ref[1:2, :], 0.0)
    o_ref[...] = _dot(x, w3_ref[...]) + v_ref[2:3, 0:4 * OUT]


def _decoder(p, fl):
    blk = 3128
    npk = NFP // 4
    mlp = p["mlp"]
    full = lambda s: pl.BlockSpec(s, lambda i: (0, 0))
    b3 = jnp.tile(mlp[2]["b"], 4)
    vecs = jnp.stack([jnp.tile(mlp[0]["b"], 4), jnp.tile(mlp[1]["b"], 4),
                      jnp.pad(b3, (0, LP - b3.shape[0]))]
                     + [jnp.zeros((LP,), jnp.float32)] * 5)
    return pl.pallas_call(
        _dec_body,
        grid=(npk // blk,),
        in_specs=[
            pl.BlockSpec((blk, LP), lambda i: (i, 0)),
            full((LP, LP)), full((LP, LP)), full((LP, 4 * OUT)),
            full((8, LP)),
        ],
        out_specs=pl.BlockSpec((blk, 4 * OUT), lambda i: (i, 0)),
        out_shape=jax.ShapeDtypeStruct((npk, 4 * OUT), jnp.float32),
        compiler_params=pltpu.CompilerParams(
            dimension_semantics=("arbitrary",)),
    )(fl, _blk4(mlp[0]["W"]), _blk4(mlp[1]["W"]), _blk4(mlp[2]["W"]), vecs)


@functools.cache
def _sc_mesh():
    return plsc.VectorSubcoreMesh(
        core_axis_name="c", subcore_axis_name="s",
        num_cores=NC, num_subcores=NS)


NK = 7


def _gather_call(table, idx3, nb):
    rows_pw = nb * BLK
    n = NW * rows_pw
    ng = nb // NK
    trows = table.shape[0]
    tstripe = trows // NS
    dt = table.dtype

    @functools.partial(
        pl.kernel,
        out_type=jax.ShapeDtypeStruct((n, L), dt),
        mesh=_sc_mesh(),
        compiler_params=pltpu.CompilerParams(use_tc_tiling_on_sc=False),
        scratch_types=[
            pltpu.VMEM((NK, BLK), jnp.int32),
            pltpu.VMEM((NK, BLK, L), dt),
            pltpu.VMEM_SHARED((trows, L), dt),
            pltpu.SemaphoreType.DMA,
            pltpu.SemaphoreType.DMA,
        ],
    )
    def k(table_ref, idx_ref, out_ref, idx_v, bufs, tab, semg, sems):
        c = lax.axis_index("c")
        s = lax.axis_index("s")
        wid = s * NC + c
        base = wid * rows_pw
        pltpu.sync_copy(table_ref.at[pl.ds(s * tstripe, tstripe)],
                        tab.at[pl.ds(s * tstripe, tstripe)])
        plsc.subcore_barrier()

        def body(g, _):
            b0 = g * NK
            di = pltpu.async_copy(idx_ref.at[wid, pl.ds(b0, NK)], idx_v,
                                  semg)
            di.wait()
            ds = [pltpu.async_copy(tab.at[idx_v.at[j]],
                                   bufs.at[j], semg) for j in range(NK)]
            for d in ds:
                d.wait()
            ss = [pltpu.async_copy(
                bufs.at[j],
                out_ref.at[pl.ds(base + (b0 + j) * BLK, BLK)], sems)
                for j in range(NK)]
            for d in ss:
                d.wait()
            return 0

        lax.fori_loop(0, ng, body, 0)
        plsc.subcore_barrier()

    return k(table, idx3)


def _scatter_call(vals_m, vals_w, idx3, zeros):
    @functools.partial(
        pl.kernel,
        out_type=jax.ShapeDtypeStruct((NC, NFP, L), jnp.float32),
        mesh=_sc_mesh(),
        compiler_params=pltpu.CompilerParams(use_tc_tiling_on_sc=False),
        scratch_types=[
            pltpu.VMEM((NK, BLK), jnp.int32),
            pltpu.VMEM((NK, BLK, L), jnp.float32),
            pltpu.VMEM_SHARED((NFP, L), jnp.float32),
            pltpu.SemaphoreType.DMA,
            pltpu.SemaphoreType.DMA,
        ],
    )
    def k(vm_ref, vw_ref, idx_ref, z_ref, out_ref, idx_v, bufs, acc,
          semr, semw):
        c = lax.axis_index("c")
        s = lax.axis_index("s")
        wid = s * NC + c
        pltpu.sync_copy(z_ref.at[pl.ds(s * STRIPE, STRIPE)],
                        acc.at[pl.ds(s * STRIPE, STRIPE)])
        plsc.subcore_barrier()

        def make_body(vref, nb, boff):
            base = wid * nb * BLK

            def body(g, _):
                b0 = g * NK
                di = pltpu.async_copy(
                    idx_ref.at[wid, pl.ds(boff + b0, NK)], idx_v, semr)
                ds = [pltpu.async_copy(
                    vref.at[pl.ds(base + (b0 + j) * BLK, BLK)],
                    bufs.at[j], semr) for j in range(NK)]
                di.wait()
                for d in ds:
                    d.wait()
                ss = [pltpu.async_copy(bufs.at[j], acc.at[idx_v.at[j]],
                                       semw, add=True) for j in range(NK)]
                for d in ss:
                    d.wait()
                return 0

            return body

        lax.fori_loop(0, NB_M // NK, make_body(vm_ref, NB_M, 0), 0)
        lax.fori_loop(0, NB_W // NK, make_body(vw_ref, NB_W, NB_M), 0)
        plsc.subcore_barrier()
        pltpu.sync_copy(acc.at[pl.ds(s * STRIPE, STRIPE)],
                        out_ref.at[c, pl.ds(s * STRIPE, STRIPE)])

    return k(vals_m, vals_w, idx3, zeros)


def kernel(fluid_node_attr, env_node_attr, mesh_edge_attr, world_edge_attr,
           params, mesh_edge_index, world_edge_index):
    p = params
    fl_attr = _pad_rows(fluid_node_attr, NFP)
    env_attr = _pad_rows(env_node_attr, NEP)
    me_attr = _pad_rows(mesh_edge_attr, EMP)
    we_attr = _pad_rows(world_edge_attr, EWP)

    ms = _pad_idx(mesh_edge_index[0], EMP, 0, NF)
    md = _pad_idx(mesh_edge_index[1], EMP, NF, NFP - NF)
    ws = _pad_idx(world_edge_index[0], EWP, 0, NE)
    wd = _pad_idx(world_edge_index[1], EWP, NF, NFP - NF)
    gidx = jnp.concatenate([ms, md, wd]).reshape(NW, NB_ALL, BLK)
    ws3 = ws.reshape(NW, NB_W, BLK)
    cidx = jnp.concatenate([md.reshape(NW, NB_M, BLK),
                            wd.reshape(NW, NB_W, BLK)], axis=1)
    zeros_nf = jnp.zeros((NFP, L), jnp.float32)

    pk = lambda x: x.reshape(-1, LP)
    unpk = lambda x: x.reshape(-1, L)
    pk3 = lambda x: x.reshape(2, -1, LP)
    mb = _mean_mat()

    fl_p = _encoder(p["node_enc"], fl_attr.reshape(-1, 48), 3128, mb)
    el_p = _encoder(p["node_enc"], env_attr.reshape(-1, 48), 2512, mb)
    me_p = _encoder(p["mesh_enc"], me_attr.reshape(-1, 28), RT_P, mb)
    we_p = _encoder(p["world_enc"], we_attr.reshape(-1, 16), RT_P, mb)

    gws = _gather_call(unpk(el_p), ws3, NB_W)
    gws_p = pk(gws)
    nbm = EMP // ROW_T
    for sp in p["steps"]:
        w1n = sp["node"]["mlp"][0]["W"]
        g = _gather_call(unpk(fl_p), gidx, NB_ALL)
        gp = pk(g)
        mpre_p, me_p = _edge_mlp(sp["mesh_edge"], me_p, (gp, 0),
                                 (gp, nbm), mb, _blk4(w1n[L:2 * L]))
        wpre_p, we_p = _edge_mlp(sp["world_edge"], we_p, (gws_p, 0),
                                 (gp, 2 * nbm), mb, _blk4(w1n[2 * L:]))
        ap = _scatter_call(unpk(mpre_p), unpk(wpre_p), cidx, zeros_nf)
        fl_p = _node_mlp(sp["node"], fl_p, pk3(ap), mb)

    return _decoder(p["decoder"], fl_p).reshape(-1, OUT)[:NF]
